# baseline (speedup 1.0000x reference)
import functools

import jax
import jax.numpy as jnp
from jax import lax
from jax.experimental import pallas as pl
from jax.experimental.pallas import tpu as pltpu
from jax.experimental.pallas import tpu_sc as plsc

MAX_LEN = 128
SIDE = MAX_LEN + 1
FLAT = SIDE * SIDE
FLAT_PAD = 16704
STRIDE = 136

_info = plsc.get_sparse_core_info()
NC = _info.num_cores
NS = _info.num_subcores
L = _info.num_lanes
NW = NC * NS


NCH = 2


def _make_lookup(B: int):
    b_per_w = B // NW
    chunk = b_per_w // NCH
    mesh = plsc.VectorSubcoreMesh(core_axis_name="c", subcore_axis_name="s")

    @functools.partial(
        pl.kernel,
        mesh=mesh,
        out_type=jax.ShapeDtypeStruct((B,), jnp.float32),
        compiler_params=pltpu.CompilerParams(
            needs_layout_passes=False, use_tc_tiling_on_sc=False
        ),
        scratch_types=[
            pltpu.VMEM((b_per_w,), jnp.int32),
            pltpu.VMEM((b_per_w,), jnp.int32),
            pltpu.VMEM((b_per_w,), jnp.int32),
            pltpu.VMEM((b_per_w,), jnp.float32),
            pltpu.VMEM_SHARED((SIDE * STRIDE,), jnp.float32),
            pltpu.VMEM((9, SIDE), jnp.float32),
            pltpu.SemaphoreType.DMA((NCH,)),
            pltpu.SemaphoreType.DMA((NCH,)),
            pltpu.SemaphoreType.DMA((NCH,)),
            pltpu.SemaphoreType.DMA,
        ],
    )
    def lookup(
        var_hbm,
        pre_hbm,
        tab_hbm,
        out_hbm,
        var_v,
        pre_v,
        idx_v,
        out_v,
        tab_s,
        blk_v,
        sem_i,
        sem_g,
        sem_o,
        sem_t,
    ):
        sid = lax.axis_index("s")
        wid = sid * NC + lax.axis_index("c")
        base = wid * b_per_w
        ins = []
        for j in range(NCH):
            lo = j * chunk
            cp_v = pltpu.make_async_copy(
                var_hbm.at[pl.ds(base + lo, chunk)],
                var_v.at[pl.ds(lo, chunk)],
                sem_i.at[j],
            )
            cp_p = pltpu.make_async_copy(
                pre_hbm.at[pl.ds(base + lo, chunk)],
                pre_v.at[pl.ds(lo, chunk)],
                sem_i.at[j],
            )
            cp_v.start()
            cp_p.start()
            ins.append((cp_v, cp_p))

        row0 = sid * 8
        nrow = 8
        cp_blk = pltpu.make_async_copy(
            tab_hbm.at[pl.ds(row0, nrow), :], blk_v.at[pl.ds(0, nrow), :], sem_t
        )
        cp_blk.start()

        @pl.when(sid == NS - 1)
        def _stage_last_row():
            pltpu.sync_copy(
                tab_hbm.at[pl.ds(SIDE - 1, 1), :], blk_v.at[pl.ds(8, 1), :]
            )

        cp_blk.wait()
        rows = []
        for k in range(nrow):
            cp_r = pltpu.make_async_copy(
                blk_v.at[k],
                tab_s.at[pl.ds((row0 + k) * STRIDE, SIDE)],
                sem_t,
            )
            cp_r.start()
            rows.append(cp_r)

        @pl.when(sid == NS - 1)
        def _bounce_last_row():
            pltpu.sync_copy(blk_v.at[8], tab_s.at[pl.ds((SIDE - 1) * STRIDE, SIDE)])

        for cp_r in rows:
            cp_r.wait()
        plsc.subcore_barrier()
        gathers = []
        for j in range(NCH):
            lo = j * chunk
            ins[j][0].wait()
            ins[j][1].wait()
            for i in range(chunk // L):
                off = lo + i * L
                v = var_v[pl.ds(off, L)]
                p = pre_v[pl.ds(off, L)]
                idx_v[pl.ds(off, L)] = v * STRIDE + p
            cp_g = pltpu.make_async_copy(
                tab_s.at[idx_v.at[pl.ds(lo, chunk)]],
                out_v.at[pl.ds(lo, chunk)],
                sem_g.at[j],
            )
            cp_g.start()
            gathers.append(cp_g)
        outs = []
        for j in range(NCH):
            lo = j * chunk
            gathers[j].wait()
            cp_o = pltpu.make_async_copy(
                out_v.at[pl.ds(lo, chunk)],
                out_hbm.at[pl.ds(base + lo, chunk)],
                sem_o.at[j],
            )
            cp_o.start()
            outs.append(cp_o)
        for cp_o in outs:
            cp_o.wait()

    return lookup


def kernel(var_len, prefix_idx, table):
    B = var_len.shape[0]
    fn = _make_lookup(B)
    return fn(
        var_len.astype(jnp.int32),
        prefix_idx.astype(jnp.int32),
        table.astype(jnp.float32),
    )


# speedup vs baseline: 1.0029x; 1.0029x over previous
import functools

import jax
import jax.numpy as jnp
from jax import lax
from jax.experimental import pallas as pl
from jax.experimental.pallas import tpu as pltpu
from jax.experimental.pallas import tpu_sc as plsc

MAX_LEN = 128
SIDE = MAX_LEN + 1
FLAT = SIDE * SIDE
FLAT_PAD = 16704
STRIDE = 136

_info = plsc.get_sparse_core_info()
NC = _info.num_cores
NS = _info.num_subcores
L = _info.num_lanes
NW = NC * NS


NCH = 2


def _make_lookup(B: int):
    b_per_w = B // NW
    chunk = b_per_w // NCH
    mesh = plsc.VectorSubcoreMesh(core_axis_name="c", subcore_axis_name="s")

    @functools.partial(
        pl.kernel,
        mesh=mesh,
        out_type=jax.ShapeDtypeStruct((B,), jnp.float32),
        compiler_params=pltpu.CompilerParams(needs_layout_passes=False),
        scratch_types=[
            pltpu.VMEM((b_per_w,), jnp.int32),
            pltpu.VMEM((b_per_w,), jnp.int32),
            pltpu.VMEM((b_per_w,), jnp.int32),
            pltpu.VMEM((b_per_w,), jnp.float32),
            pltpu.VMEM_SHARED((FLAT,), jnp.float32),
            pltpu.SemaphoreType.DMA((NCH,)),
            pltpu.SemaphoreType.DMA((NCH,)),
            pltpu.SemaphoreType.DMA((NCH,)),
            pltpu.SemaphoreType.DMA,
        ],
    )
    def lookup(
        var_hbm,
        pre_hbm,
        tab_hbm,
        out_hbm,
        var_v,
        pre_v,
        idx_v,
        out_v,
        tab_s,
        sem_i,
        sem_g,
        sem_o,
        sem_t,
    ):
        sid = lax.axis_index("s")
        wid = sid * NC + lax.axis_index("c")
        base = wid * b_per_w
        ins = []
        for j in range(NCH):
            lo = j * chunk
            cp_v = pltpu.make_async_copy(
                var_hbm.at[pl.ds(base + lo, chunk)],
                var_v.at[pl.ds(lo, chunk)],
                sem_i.at[j],
            )
            cp_p = pltpu.make_async_copy(
                pre_hbm.at[pl.ds(base + lo, chunk)],
                pre_v.at[pl.ds(lo, chunk)],
                sem_i.at[j],
            )
            cp_v.start()
            cp_p.start()
            ins.append((cp_v, cp_p))

        @pl.when(sid == 0)
        def _stage_table():
            cp_t = pltpu.make_async_copy(tab_hbm, tab_s, sem_t)
            cp_t.start()
            cp_t.wait()

        plsc.subcore_barrier()
        gathers = []
        for j in range(NCH):
            lo = j * chunk
            ins[j][0].wait()
            ins[j][1].wait()
            for i in range(chunk // L):
                off = lo + i * L
                v = var_v[pl.ds(off, L)]
                p = pre_v[pl.ds(off, L)]
                idx_v[pl.ds(off, L)] = v * SIDE + p
            cp_g = pltpu.make_async_copy(
                tab_s.at[idx_v.at[pl.ds(lo, chunk)]],
                out_v.at[pl.ds(lo, chunk)],
                sem_g.at[j],
            )
            cp_g.start()
            gathers.append(cp_g)
        outs = []
        for j in range(NCH):
            lo = j * chunk
            gathers[j].wait()
            cp_o = pltpu.make_async_copy(
                out_v.at[pl.ds(lo, chunk)],
                out_hbm.at[pl.ds(base + lo, chunk)],
                sem_o.at[j],
            )
            cp_o.start()
            outs.append(cp_o)
        for cp_o in outs:
            cp_o.wait()

    return lookup


def kernel(var_len, prefix_idx, table):
    B = var_len.shape[0]
    fn = _make_lookup(B)
    return fn(
        var_len.astype(jnp.int32),
        prefix_idx.astype(jnp.int32),
        table.reshape(-1).astype(jnp.float32),
    )


# speedup vs baseline: 1.0049x; 1.0020x over previous
import functools

import jax
import jax.numpy as jnp
from jax import lax
from jax.experimental import pallas as pl
from jax.experimental.pallas import tpu as pltpu
from jax.experimental.pallas import tpu_sc as plsc

MAX_LEN = 128
SIDE = MAX_LEN + 1
FLAT = SIDE * SIDE
FLAT_PAD = 16704
STRIDE = 136

_info = plsc.get_sparse_core_info()
NC = _info.num_cores
NS = _info.num_subcores
L = _info.num_lanes
NW = NC * NS


NCH = 2


def _make_lookup(B: int):
    b_per_w = B // NW
    chunk = b_per_w // NCH
    mesh = plsc.VectorSubcoreMesh(core_axis_name="c", subcore_axis_name="s")

    @functools.partial(
        pl.kernel,
        mesh=mesh,
        out_type=jax.ShapeDtypeStruct((B,), jnp.float32),
        compiler_params=pltpu.CompilerParams(needs_layout_passes=False),
        scratch_types=[
            pltpu.VMEM((b_per_w,), jnp.int32),
            pltpu.VMEM((b_per_w,), jnp.int32),
            pltpu.VMEM((b_per_w,), jnp.int32),
            pltpu.VMEM((b_per_w,), jnp.float32),
            pltpu.VMEM_SHARED((FLAT,), jnp.float32),
            pltpu.SemaphoreType.DMA((NCH,)),
            pltpu.SemaphoreType.DMA((NCH,)),
            pltpu.SemaphoreType.DMA((NCH,)),
            pltpu.SemaphoreType.DMA,
        ],
    )
    def lookup(
        var_hbm,
        pre_hbm,
        tab_hbm,
        out_hbm,
        var_v,
        pre_v,
        idx_v,
        out_v,
        tab_s,
        sem_i,
        sem_g,
        sem_o,
        sem_t,
    ):
        sid = lax.axis_index("s")
        wid = sid * NC + lax.axis_index("c")
        base = wid * b_per_w
        ins = []
        for j in range(NCH):
            lo = j * chunk
            cp_v = pltpu.make_async_copy(
                var_hbm.at[pl.ds(base + lo, chunk)],
                var_v.at[pl.ds(lo, chunk)],
                sem_i.at[j],
            )
            cp_p = pltpu.make_async_copy(
                pre_hbm.at[pl.ds(base + lo, chunk)],
                pre_v.at[pl.ds(lo, chunk)],
                sem_i.at[j],
            )
            cp_v.start()
            cp_p.start()
            ins.append((cp_v, cp_p))

        cp_t = pltpu.make_async_copy(tab_hbm, tab_s, sem_t)

        @pl.when(sid == 0)
        def _stage_table_start():
            cp_t.start()

        for j in range(NCH):
            lo = j * chunk
            ins[j][0].wait()
            ins[j][1].wait()
            for i in range(chunk // L):
                off = lo + i * L
                v = var_v[pl.ds(off, L)]
                p = pre_v[pl.ds(off, L)]
                idx_v[pl.ds(off, L)] = v * SIDE + p

        @pl.when(sid == 0)
        def _stage_table_wait():
            cp_t.wait()

        plsc.subcore_barrier()
        gathers = []
        for j in range(NCH):
            lo = j * chunk
            cp_g = pltpu.make_async_copy(
                tab_s.at[idx_v.at[pl.ds(lo, chunk)]],
                out_v.at[pl.ds(lo, chunk)],
                sem_g.at[j],
            )
            cp_g.start()
            gathers.append(cp_g)
        outs = []
        for j in range(NCH):
            lo = j * chunk
            gathers[j].wait()
            cp_o = pltpu.make_async_copy(
                out_v.at[pl.ds(lo, chunk)],
                out_hbm.at[pl.ds(base + lo, chunk)],
                sem_o.at[j],
            )
            cp_o.start()
            outs.append(cp_o)
        for cp_o in outs:
            cp_o.wait()

    return lookup


def kernel(var_len, prefix_idx, table):
    B = var_len.shape[0]
    fn = _make_lookup(B)
    return fn(
        var_len.astype(jnp.int32),
        prefix_idx.astype(jnp.int32),
        table.reshape(-1).astype(jnp.float32),
    )


# speedup vs baseline: 1.0121x; 1.0072x over previous
import functools

import jax
import jax.numpy as jnp
from jax import lax
from jax.experimental import pallas as pl
from jax.experimental.pallas import tpu as pltpu
from jax.experimental.pallas import tpu_sc as plsc

MAX_LEN = 128
SIDE = MAX_LEN + 1
FLAT = SIDE * SIDE
FLAT_PAD = 16704
STRIDE = 136

_info = plsc.get_sparse_core_info()
NC = _info.num_cores
NS = _info.num_subcores
L = _info.num_lanes
NW = NC * NS


NCH = 1


def _make_lookup(B: int):
    b_per_w = B // NW
    chunk = b_per_w // NCH
    mesh = plsc.VectorSubcoreMesh(core_axis_name="c", subcore_axis_name="s")

    @functools.partial(
        pl.kernel,
        mesh=mesh,
        out_type=jax.ShapeDtypeStruct((B,), jnp.float32),
        compiler_params=pltpu.CompilerParams(needs_layout_passes=False),
        scratch_types=[
            pltpu.VMEM((b_per_w,), jnp.int32),
            pltpu.VMEM((b_per_w,), jnp.int32),
            pltpu.VMEM((b_per_w,), jnp.int32),
            pltpu.VMEM((b_per_w,), jnp.float32),
            pltpu.VMEM_SHARED((FLAT,), jnp.float32),
            pltpu.SemaphoreType.DMA((NCH,)),
            pltpu.SemaphoreType.DMA((NCH,)),
            pltpu.SemaphoreType.DMA((NCH,)),
            pltpu.SemaphoreType.DMA,
        ],
    )
    def lookup(
        var_hbm,
        pre_hbm,
        tab_hbm,
        out_hbm,
        var_v,
        pre_v,
        idx_v,
        out_v,
        tab_s,
        sem_i,
        sem_g,
        sem_o,
        sem_t,
    ):
        sid = lax.axis_index("s")
        wid = sid * NC + lax.axis_index("c")
        base = wid * b_per_w
        ins = []
        for j in range(NCH):
            lo = j * chunk
            cp_v = pltpu.make_async_copy(
                var_hbm.at[pl.ds(base + lo, chunk)],
                var_v.at[pl.ds(lo, chunk)],
                sem_i.at[j],
            )
            cp_p = pltpu.make_async_copy(
                pre_hbm.at[pl.ds(base + lo, chunk)],
                pre_v.at[pl.ds(lo, chunk)],
                sem_i.at[j],
            )
            cp_v.start()
            cp_p.start()
            ins.append((cp_v, cp_p))

        cp_t = pltpu.make_async_copy(tab_hbm, tab_s, sem_t)

        @pl.when(sid == 0)
        def _stage_table_start():
            cp_t.start()

        for j in range(NCH):
            lo = j * chunk
            ins[j][0].wait()
            ins[j][1].wait()
            for i in range(chunk // L):
                off = lo + i * L
                v = var_v[pl.ds(off, L)]
                p = pre_v[pl.ds(off, L)]
                idx_v[pl.ds(off, L)] = v * SIDE + p

        @pl.when(sid == 0)
        def _stage_table_wait():
            cp_t.wait()

        plsc.subcore_barrier()
        gathers = []
        for j in range(NCH):
            lo = j * chunk
            cp_g = pltpu.make_async_copy(
                tab_s.at[idx_v.at[pl.ds(lo, chunk)]],
                out_v.at[pl.ds(lo, chunk)],
                sem_g.at[j],
            )
            cp_g.start()
            gathers.append(cp_g)
        outs = []
        for j in range(NCH):
            lo = j * chunk
            gathers[j].wait()
            cp_o = pltpu.make_async_copy(
                out_v.at[pl.ds(lo, chunk)],
                out_hbm.at[pl.ds(base + lo, chunk)],
                sem_o.at[j],
            )
            cp_o.start()
            outs.append(cp_o)
        for cp_o in outs:
            cp_o.wait()

    return lookup


def kernel(var_len, prefix_idx, table):
    B = var_len.shape[0]
    fn = _make_lookup(B)
    return fn(
        var_len.astype(jnp.int32),
        prefix_idx.astype(jnp.int32),
        table.reshape(-1).astype(jnp.float32),
    )


# speedup vs baseline: 1.0186x; 1.0064x over previous
import functools

import jax
import jax.numpy as jnp
from jax import lax
from jax.experimental import pallas as pl
from jax.experimental.pallas import tpu as pltpu
from jax.experimental.pallas import tpu_sc as plsc

MAX_LEN = 128
SIDE = MAX_LEN + 1
FLAT = SIDE * SIDE
FLAT_PAD = 16704
STRIDE = 136

_info = plsc.get_sparse_core_info()
NC = _info.num_cores
NS = _info.num_subcores
L = _info.num_lanes
NW = NC * NS


NCH = 1
GCH = 2


def _make_lookup(B: int):
    b_per_w = B // NW
    chunk = b_per_w // NCH
    mesh = plsc.VectorSubcoreMesh(core_axis_name="c", subcore_axis_name="s")

    @functools.partial(
        pl.kernel,
        mesh=mesh,
        out_type=jax.ShapeDtypeStruct((B,), jnp.float32),
        compiler_params=pltpu.CompilerParams(needs_layout_passes=False),
        scratch_types=[
            pltpu.VMEM((b_per_w,), jnp.int32),
            pltpu.VMEM((b_per_w,), jnp.int32),
            pltpu.VMEM((b_per_w,), jnp.int32),
            pltpu.VMEM((b_per_w,), jnp.float32),
            pltpu.VMEM_SHARED((FLAT,), jnp.float32),
            pltpu.SemaphoreType.DMA((NCH,)),
            pltpu.SemaphoreType.DMA((GCH,)),
            pltpu.SemaphoreType.DMA((GCH,)),
            pltpu.SemaphoreType.DMA,
        ],
    )
    def lookup(
        var_hbm,
        pre_hbm,
        tab_hbm,
        out_hbm,
        var_v,
        pre_v,
        idx_v,
        out_v,
        tab_s,
        sem_i,
        sem_g,
        sem_o,
        sem_t,
    ):
        sid = lax.axis_index("s")
        wid = sid * NC + lax.axis_index("c")
        base = wid * b_per_w
        ins = []
        for j in range(NCH):
            lo = j * chunk
            cp_v = pltpu.make_async_copy(
                var_hbm.at[pl.ds(base + lo, chunk)],
                var_v.at[pl.ds(lo, chunk)],
                sem_i.at[j],
            )
            cp_p = pltpu.make_async_copy(
                pre_hbm.at[pl.ds(base + lo, chunk)],
                pre_v.at[pl.ds(lo, chunk)],
                sem_i.at[j],
            )
            cp_v.start()
            cp_p.start()
            ins.append((cp_v, cp_p))

        cp_t = pltpu.make_async_copy(tab_hbm, tab_s, sem_t)

        @pl.when(sid == 0)
        def _stage_table_start():
            cp_t.start()

        for j in range(NCH):
            lo = j * chunk
            ins[j][0].wait()
            ins[j][1].wait()
            for i in range(chunk // L):
                off = lo + i * L
                v = var_v[pl.ds(off, L)]
                p = pre_v[pl.ds(off, L)]
                idx_v[pl.ds(off, L)] = v * SIDE + p

        @pl.when(sid == 0)
        def _stage_table_wait():
            cp_t.wait()

        plsc.subcore_barrier()
        gchunk = b_per_w // GCH
        gathers = []
        for j in range(GCH):
            lo = j * gchunk
            cp_g = pltpu.make_async_copy(
                tab_s.at[idx_v.at[pl.ds(lo, gchunk)]],
                out_v.at[pl.ds(lo, gchunk)],
                sem_g.at[j],
            )
            cp_g.start()
            gathers.append(cp_g)
        outs = []
        for j in range(GCH):
            lo = j * gchunk
            gathers[j].wait()
            cp_o = pltpu.make_async_copy(
                out_v.at[pl.ds(lo, gchunk)],
                out_hbm.at[pl.ds(base + lo, gchunk)],
                sem_o.at[j],
            )
            cp_o.start()
            outs.append(cp_o)
        for cp_o in outs:
            cp_o.wait()

    return lookup


def kernel(var_len, prefix_idx, table):
    B = var_len.shape[0]
    fn = _make_lookup(B)
    return fn(
        var_len.astype(jnp.int32),
        prefix_idx.astype(jnp.int32),
        table.reshape(-1).astype(jnp.float32),
    )
